# baseline (speedup 1.0000x reference)
import jax
import jax.numpy as jnp
from jax.experimental import pallas as pl
from jax.experimental.pallas import tpu as pltpu

_CH = 512
_NTOK = 8


def _body_factory(B, S, D):
    nch = S // _CH
    trows = B * S // _NTOK

    def body(w_hbm, tok_hbm, pos_hbm, tokout_hbm, wbuf, sem_in, sem_out, sem_tok):
        for t in range(_NTOK):
            pltpu.make_async_copy(
                tok_hbm.at[pl.ds(t * trows, trows)],
                tokout_hbm.at[pl.ds(t * trows, trows)],
                sem_tok,
            ).start()

        def load(c):
            pltpu.make_async_copy(
                w_hbm.at[pl.ds(c * _CH, _CH)], wbuf.at[c % 2], sem_in
            ).start()

        def store(c, b):
            return pltpu.make_async_copy(
                wbuf.at[c % 2], pos_hbm.at[b, pl.ds(c * _CH, _CH)],
                sem_out.at[c % 2],
            )

        load(0)
        for c in range(nch):
            pltpu.make_async_copy(
                w_hbm.at[pl.ds(c * _CH, _CH)], wbuf.at[c % 2], sem_in
            ).wait()
            if c + 1 < nch:
                load(c + 1)
            if c >= 2:
                for b in range(B):
                    store(c - 2, b).wait()
            for b in range(B):
                store(c, b).start()
        for c in (nch - 2, nch - 1):
            for b in range(B):
                store(c, b).wait()
        for t in range(_NTOK):
            pltpu.make_async_copy(
                tok_hbm.at[pl.ds(t * trows, trows)],
                tokout_hbm.at[pl.ds(t * trows, trows)],
                sem_tok,
            ).wait()

    return body


def kernel(tokens, token_embed, W_pos):
    B, S, D = token_embed.shape
    tok_flat = token_embed.reshape(B * S, D)
    pos, tok = pl.pallas_call(
        _body_factory(B, S, D),
        in_specs=[
            pl.BlockSpec(memory_space=pl.ANY),
            pl.BlockSpec(memory_space=pl.ANY),
        ],
        out_specs=[
            pl.BlockSpec(memory_space=pl.ANY),
            pl.BlockSpec(memory_space=pl.ANY),
        ],
        out_shape=[
            jax.ShapeDtypeStruct((B, S, D), W_pos.dtype),
            jax.ShapeDtypeStruct((B * S, D), token_embed.dtype),
        ],
        scratch_shapes=[
            pltpu.VMEM((2, _CH, D), W_pos.dtype),
            pltpu.SemaphoreType.DMA,
            pltpu.SemaphoreType.DMA((2,)),
            pltpu.SemaphoreType.DMA,
        ],
    )(W_pos, tok_flat)
    return (pos, tok.reshape(B, S, D))


# speedup vs baseline: 29.2878x; 29.2878x over previous
import jax
import jax.numpy as jnp
from jax.experimental import pallas as pl


def _body(w_ref, t_ref, pos_ref, tok_ref):
    pos_ref[...] = w_ref[...][None, :, :]
    tok_ref[...] = t_ref[...]


def kernel(tokens, token_embed, W_pos):
    B, S, D = token_embed.shape
    CH = 1024
    tok_flat = token_embed.reshape(B * S, D)
    pos, tok = pl.pallas_call(
        _body,
        grid=(S // CH, B),
        in_specs=[
            pl.BlockSpec((CH, D), lambda i, j: (i, 0)),
            pl.BlockSpec((CH, D), lambda i, j: (i * B + j, 0)),
        ],
        out_specs=[
            pl.BlockSpec((1, CH, D), lambda i, j: (j, i, 0)),
            pl.BlockSpec((CH, D), lambda i, j: (i * B + j, 0)),
        ],
        out_shape=[
            jax.ShapeDtypeStruct((B, S, D), W_pos.dtype),
            jax.ShapeDtypeStruct((B * S, D), token_embed.dtype),
        ],
    )(W_pos, tok_flat)
    return (pos, tok.reshape(B, S, D))


# speedup vs baseline: 30.2556x; 1.0330x over previous
import jax
import jax.numpy as jnp
from jax.experimental import pallas as pl


def _body(w_ref, t_ref, tok_ref, pos_ref):
    tok_ref[...] = t_ref[...]
    pos_ref[...] = jnp.broadcast_to(w_ref[...][None, :, :], pos_ref.shape)


def kernel(tokens, token_embed, W_pos):
    B, S, D = token_embed.shape
    CH = 512
    TCH = CH * B
    tok_flat = token_embed.reshape(B * S, D)
    tok, pos = pl.pallas_call(
        _body,
        grid=(S // CH,),
        in_specs=[
            pl.BlockSpec((CH, D), lambda i: (i, 0)),
            pl.BlockSpec((TCH, D), lambda i: (i, 0)),
        ],
        out_specs=[
            pl.BlockSpec((TCH, D), lambda i: (i, 0)),
            pl.BlockSpec((B, CH, D), lambda i: (0, i, 0)),
        ],
        out_shape=[
            jax.ShapeDtypeStruct((B * S, D), token_embed.dtype),
            jax.ShapeDtypeStruct((B, S, D), W_pos.dtype),
        ],
    )(W_pos, tok_flat)
    return (pos, tok.reshape(B, S, D))
